# Initial kernel scaffold; baseline (speedup 1.0000x reference)
import jax
import jax.numpy as jnp
from jax.experimental import pallas as pl


def kernel(x, edge_index, Wl1, Wr1, a1, b1, Wres1, Wl2, Wr2, a2, b2, Wres2, l1W, l1b, g1, be1, l2W, l2b, g2, be2, l3W, l3b):
    raise NotImplementedError("write your pallas kernel here")


# speedup vs baseline: 3.0866x; 3.0866x over previous
import functools

import jax
import jax.numpy as jnp
from jax import lax
from jax.experimental import pallas as pl
from jax.experimental.pallas import tpu as pltpu
from jax.experimental.pallas import tpu_sc as plsc

N_NODES = 1024
N_EDGES = 8192
HEADS = 20

_F32 = jnp.float32
_DN_T = (((1,), (1,)), ((), ()))
_DN_N = (((1,), (0,)), ((), ()))


def _blk(dim, target):
    if dim <= target:
        return dim
    b = (target // 128) * 128
    while b > 0 and dim % b:
        b -= 128
    return b if b > 0 else dim


def _sc_gather_rows(table, idx):
    n, d = table.shape
    e = idx.shape[0]
    nw = 32
    bpw = e // nw
    ch = max(8, ((96 * 1024) // (d * 4)) // 8 * 8)
    while ch > 8 and (bpw % ch or (bpw // ch) % 2):
        ch -= 8
    nbuf = 2 if 2 * ch * d + bpw <= 120000 else 1
    n_chunks = bpw // ch
    mesh = plsc.VectorSubcoreMesh(core_axis_name="c", subcore_axis_name="s")
    scratch = (
        [pltpu.VMEM((bpw,), jnp.int32)]
        + [pltpu.VMEM((ch, d), _F32) for _ in range(nbuf)]
        + [pltpu.SemaphoreType.DMA for _ in range(nbuf)]
    )

    @functools.partial(pl.kernel,
                       out_type=jax.ShapeDtypeStruct((e, d), _F32),
                       mesh=mesh, scratch_types=scratch)
    def gk(tab_h, idx_h, out_h, idx_v, *bufsem):
        bufs = bufsem[:nbuf]
        sems = bufsem[nbuf:]
        wid = lax.axis_index("s") * 2 + lax.axis_index("c")
        base = wid * bpw
        pltpu.sync_copy(idx_h.at[pl.ds(base, bpw)], idx_v)

        def gather(g, slot):
            pltpu.make_async_copy(
                tab_h.at[idx_v.at[pl.ds(g * ch, ch)]], bufs[slot], sems[slot]
            ).start()

        def drain(g, slot):
            pltpu.make_async_copy(
                tab_h.at[idx_v.at[pl.ds(g * ch, ch)]], bufs[slot], sems[slot]
            ).wait()
            pltpu.sync_copy(bufs[slot], out_h.at[pl.ds(base + g * ch, ch)])

        if nbuf == 1:
            def body1(g, carry):
                gather(g, 0)
                drain(g, 0)
                return carry

            lax.fori_loop(0, n_chunks, body1, 0)
        else:
            n_pairs = n_chunks // 2
            gather(0, 0)

            def body2(h, carry):
                g0 = 2 * h
                gather(g0 + 1, 1)
                drain(g0, 0)

                @pl.when(h + 1 < n_pairs)
                def _():
                    gather(g0 + 2, 0)

                drain(g0 + 1, 1)
                return carry

            lax.fori_loop(0, n_pairs, body2, 0)

    return gk(table, idx)


def _sc_gather4(glr, gli, grr, gri, src, dst):
    return (_sc_gather_rows(glr, src), _sc_gather_rows(gli, src),
            _sc_gather_rows(grr, dst), _sc_gather_rows(gri, dst))


def _cmm_big(xr, xi, w0, w1, bias=None, relu=False):
    m, k_dim = xr.shape
    nout = w0.shape[0]
    bm = _blk(m, 256)
    bn = _blk(nout, 512)
    bk = _blk(k_dim, 2048)
    gm, gn, gk = m // bm, nout // bn, k_dim // bk
    has_xi = xi is not None
    has_b = bias is not None

    def body(*refs):
        i = 0
        xr_ref = refs[i]; i += 1
        xi_ref = None
        if has_xi:
            xi_ref = refs[i]; i += 1
        w0_ref = refs[i]; w1_ref = refs[i + 1]; i += 2
        b_ref = None
        if has_b:
            b_ref = refs[i]; i += 1
        or_ref, oi_ref, accr, acci = refs[i:i + 4]
        kk = pl.program_id(2)

        @pl.when(kk == 0)
        def _():
            accr[...] = jnp.zeros(accr.shape, _F32)
            acci[...] = jnp.zeros(acci.shape, _F32)

        xrb = xr_ref[...]
        w0b = w0_ref[...]
        w1b = w1_ref[...]
        pr = lax.dot_general(xrb, w0b, _DN_T, preferred_element_type=_F32)
        pi = lax.dot_general(xrb, w1b, _DN_T, preferred_element_type=_F32)
        if has_xi:
            xib = xi_ref[...]
            pr = pr - lax.dot_general(xib, w1b, _DN_T, preferred_element_type=_F32)
            pi = pi + lax.dot_general(xib, w0b, _DN_T, preferred_element_type=_F32)
        accr[...] += pr
        acci[...] += pi

        @pl.when(kk == gk - 1)
        def _():
            yr = accr[...]
            yi = acci[...]
            if has_b:
                yr = yr + b_ref[0]
                yi = yi + b_ref[1]
            if relu:
                yr = jnp.maximum(yr, 0.0)
                yi = jnp.maximum(yi, 0.0)
            or_ref[...] = yr
            oi_ref[...] = yi

    in_specs = [pl.BlockSpec((bm, bk), lambda m_, n_, k_: (m_, k_))]
    ops = [xr]
    if has_xi:
        in_specs.append(pl.BlockSpec((bm, bk), lambda m_, n_, k_: (m_, k_)))
        ops.append(xi)
    in_specs += [pl.BlockSpec((bn, bk), lambda m_, n_, k_: (n_, k_))] * 2
    ops += [w0, w1]
    if has_b:
        in_specs.append(pl.BlockSpec((2, bn), lambda m_, n_, k_: (0, n_)))
        ops.append(bias)
    yr, yi = pl.pallas_call(
        body,
        grid=(gm, gn, gk),
        in_specs=in_specs,
        out_specs=[pl.BlockSpec((bm, bn), lambda m_, n_, k_: (m_, n_))] * 2,
        out_shape=[jax.ShapeDtypeStruct((m, nout), _F32)] * 2,
        scratch_shapes=[pltpu.VMEM((bm, bn), _F32)] * 2,
        compiler_params=pltpu.CompilerParams(
            dimension_semantics=("parallel", "parallel", "arbitrary")),
    )(*ops)
    return yr, yi


def _cmm_mlp(xr, xi, w0, w1, bias, mode, g=None, be=None):
    m, k_dim = xr.shape
    nout = w0.shape[0]
    bk = _blk(k_dim, 1024)
    gk = k_dim // bk

    def body(*refs):
        i = 0
        xr_ref, xi_ref, w0_ref, w1_ref, b_ref = refs[:5]
        i = 5
        g_ref = be_ref = None
        if mode == "bn_relu":
            g_ref, be_ref = refs[i:i + 2]
            i += 2
        or_ref, oi_ref, accr, acci = refs[i:i + 4]
        kk = pl.program_id(0)

        @pl.when(kk == 0)
        def _():
            accr[...] = jnp.zeros(accr.shape, _F32)
            acci[...] = jnp.zeros(acci.shape, _F32)

        xrb = xr_ref[...]
        xib = xi_ref[...]
        w0b = w0_ref[...]
        w1b = w1_ref[...]
        accr[...] += (lax.dot_general(xrb, w0b, _DN_T, preferred_element_type=_F32)
                      - lax.dot_general(xib, w1b, _DN_T, preferred_element_type=_F32))
        acci[...] += (lax.dot_general(xrb, w1b, _DN_T, preferred_element_type=_F32)
                      + lax.dot_general(xib, w0b, _DN_T, preferred_element_type=_F32))

        @pl.when(kk == gk - 1)
        def _():
            yr = accr[...] + b_ref[0]
            yi = acci[...] + b_ref[1]
            if mode == "bn_relu":
                mr = jnp.mean(yr, axis=0, keepdims=True)
                vr = jnp.mean((yr - mr) ** 2, axis=0, keepdims=True)
                yr = (yr - mr) * lax.rsqrt(vr + 1e-5) * g_ref[0] + be_ref[0]
                mi = jnp.mean(yi, axis=0, keepdims=True)
                vi = jnp.mean((yi - mi) ** 2, axis=0, keepdims=True)
                yi = (yi - mi) * lax.rsqrt(vi + 1e-5) * g_ref[1] + be_ref[1]
                yr = jnp.maximum(yr, 0.0)
                yi = jnp.maximum(yi, 0.0)
            elif mode == "norm":
                nrm = jnp.sqrt(jnp.sum(yr * yr + yi * yi, axis=1, keepdims=True))
                dnm = jnp.maximum(nrm, 1.0)
                yr = yr / dnm
                yi = yi / dnm
            or_ref[...] = yr
            oi_ref[...] = yi

    in_specs = [
        pl.BlockSpec((m, bk), lambda k_: (0, k_)),
        pl.BlockSpec((m, bk), lambda k_: (0, k_)),
        pl.BlockSpec((nout, bk), lambda k_: (0, k_)),
        pl.BlockSpec((nout, bk), lambda k_: (0, k_)),
        pl.BlockSpec((2, nout), lambda k_: (0, 0)),
    ]
    ops = [xr, xi, w0, w1, bias]
    if mode == "bn_relu":
        in_specs += [pl.BlockSpec((2, nout), lambda k_: (0, 0))] * 2
        ops += [g, be]
    yr, yi = pl.pallas_call(
        body,
        grid=(gk,),
        in_specs=in_specs,
        out_specs=[pl.BlockSpec((m, nout), lambda k_: (0, 0))] * 2,
        out_shape=[jax.ShapeDtypeStruct((m, nout), _F32)] * 2,
        scratch_shapes=[pltpu.VMEM((m, nout), _F32)] * 2,
        compiler_params=pltpu.CompilerParams(
            dimension_semantics=("arbitrary",)),
    )(*ops)
    return yr, yi


def _er(slr, sli, srr, sri, a0bd, a1bdn):
    e, hc = slr.shape
    be = 512
    bk = _blk(hc, 2560)
    ge, gk = e // be, hc // bk

    def body(slr_ref, sli_ref, srr_ref, sri_ref, a0_ref, a1_ref, out_ref, acc):
        kk = pl.program_id(1)

        @pl.when(kk == 0)
        def _():
            acc[...] = jnp.zeros(acc.shape, _F32)

        hr = jnp.maximum(slr_ref[...] + srr_ref[...], 0.0)
        hi = jnp.maximum(sli_ref[...] + sri_ref[...], 0.0)
        acc[...] += (lax.dot_general(hr, a0_ref[...], _DN_N, preferred_element_type=_F32)
                     + lax.dot_general(hi, a1_ref[...], _DN_N, preferred_element_type=_F32))

        @pl.when(kk == gk - 1)
        def _():
            out_ref[...] = acc[...]

    return pl.pallas_call(
        body,
        grid=(ge, gk),
        in_specs=[pl.BlockSpec((be, bk), lambda e_, k_: (e_, k_))] * 4
        + [pl.BlockSpec((bk, HEADS), lambda e_, k_: (k_, 0))] * 2,
        out_specs=pl.BlockSpec((be, HEADS), lambda e_, k_: (e_, 0)),
        out_shape=jax.ShapeDtypeStruct((e, HEADS), _F32),
        scratch_shapes=[pltpu.VMEM((be, HEADS), _F32)],
        compiler_params=pltpu.CompilerParams(
            dimension_semantics=("parallel", "arbitrary")),
    )(slr, sli, srr, sri, a0bd, a1bdn)


def _softnum(er):
    def body(er_ref, out_ref):
        erb = er_ref[...]
        mg = jnp.max(erb, axis=0, keepdims=True)
        out_ref[...] = jnp.exp(erb - mg)

    return pl.pallas_call(
        body,
        out_shape=jax.ShapeDtypeStruct(er.shape, _F32),
    )(er)


def _den(ex, dst2d):
    e, h = ex.shape
    be = 512
    ge = e // be

    def body(dst_ref, ex_ref, out_ref, acc):
        ee = pl.program_id(0)

        @pl.when(ee == 0)
        def _():
            acc[...] = jnp.zeros(acc.shape, _F32)

        iota = lax.broadcasted_iota(jnp.int32, (N_NODES, be), 0)
        oh = (iota == dst_ref[...]).astype(_F32)
        acc[...] += lax.dot_general(oh, ex_ref[...], _DN_N,
                                    preferred_element_type=_F32)

        @pl.when(ee == ge - 1)
        def _():
            out_ref[...] = acc[...]

    return pl.pallas_call(
        body,
        grid=(ge,),
        in_specs=[
            pl.BlockSpec((1, be), lambda e_: (0, e_)),
            pl.BlockSpec((be, h), lambda e_: (e_, 0)),
        ],
        out_specs=pl.BlockSpec((N_NODES, h), lambda e_: (0, 0)),
        out_shape=jax.ShapeDtypeStruct((N_NODES, h), _F32),
        scratch_shapes=[pltpu.VMEM((N_NODES, h), _F32)],
        compiler_params=pltpu.CompilerParams(
            dimension_semantics=("arbitrary",)),
    )(dst2d, ex)


def _agg(ex, slr, sli, dst2d, bh):
    e, hc = slr.shape
    bc = _blk(hc, 512)
    be = 512
    gc, ge = hc // bc, e // be

    def body(dst_ref, ex_ref, bh_ref, slr_ref, sli_ref, or_ref, oi_ref,
             accr, acci):
        ee = pl.program_id(1)

        @pl.when(ee == 0)
        def _():
            accr[...] = jnp.zeros(accr.shape, _F32)
            acci[...] = jnp.zeros(acci.shape, _F32)

        iota = lax.broadcasted_iota(jnp.int32, (N_NODES, be), 0)
        oh = (iota == dst_ref[...]).astype(_F32)
        exc = lax.dot_general(ex_ref[...], bh_ref[...], _DN_N,
                              preferred_element_type=_F32)
        accr[...] += lax.dot_general(oh, exc * slr_ref[...], _DN_N,
                                     preferred_element_type=_F32)
        acci[...] += lax.dot_general(oh, exc * sli_ref[...], _DN_N,
                                     preferred_element_type=_F32)

        @pl.when(ee == ge - 1)
        def _():
            or_ref[...] = accr[...]
            oi_ref[...] = acci[...]

    return pl.pallas_call(
        body,
        grid=(gc, ge),
        in_specs=[
            pl.BlockSpec((1, be), lambda c_, e_: (0, e_)),
            pl.BlockSpec((be, HEADS), lambda c_, e_: (e_, 0)),
            pl.BlockSpec((HEADS, bc), lambda c_, e_: (0, c_)),
            pl.BlockSpec((be, bc), lambda c_, e_: (e_, c_)),
            pl.BlockSpec((be, bc), lambda c_, e_: (e_, c_)),
        ],
        out_specs=[pl.BlockSpec((N_NODES, bc), lambda c_, e_: (0, c_))] * 2,
        out_shape=[jax.ShapeDtypeStruct((N_NODES, hc), _F32)] * 2,
        scratch_shapes=[pltpu.VMEM((N_NODES, bc), _F32)] * 2,
        compiler_params=pltpu.CompilerParams(
            dimension_semantics=("parallel", "arbitrary")),
    )(dst2d, ex, bh, slr, sli)


def _post(aggr, aggi, den, bh, resr, resi, bias):
    n, hc = aggr.shape
    bc = _blk(hc, 512)
    gc = hc // bc

    def body(aggr_ref, aggi_ref, den_ref, bh_ref, resr_ref, resi_ref, b_ref,
             or_ref, oi_ref):
        dexp = lax.dot_general(den_ref[...], bh_ref[...], _DN_N,
                               preferred_element_type=_F32)
        r = 1.0 / (dexp + 1e-30)
        or_ref[...] = jnp.maximum(aggr_ref[...] * r + resr_ref[...] + b_ref[0], 0.0)
        oi_ref[...] = jnp.maximum(aggi_ref[...] * r + resi_ref[...] + b_ref[1], 0.0)

    return pl.pallas_call(
        body,
        grid=(gc,),
        in_specs=[
            pl.BlockSpec((n, bc), lambda c_: (0, c_)),
            pl.BlockSpec((n, bc), lambda c_: (0, c_)),
            pl.BlockSpec((n, HEADS), lambda c_: (0, 0)),
            pl.BlockSpec((HEADS, bc), lambda c_: (0, c_)),
            pl.BlockSpec((n, bc), lambda c_: (0, c_)),
            pl.BlockSpec((n, bc), lambda c_: (0, c_)),
            pl.BlockSpec((2, bc), lambda c_: (0, c_)),
        ],
        out_specs=[pl.BlockSpec((n, bc), lambda c_: (0, c_))] * 2,
        out_shape=[jax.ShapeDtypeStruct((n, hc), _F32)] * 2,
        compiler_params=pltpu.CompilerParams(
            dimension_semantics=("parallel",)),
    )(aggr, aggi, den, bh, resr, resi, bias)


def _gat_layer(xr, xi, src, dst, dst2d, wl, wr, att, bias, wres, ch):
    if xi is None:
        glr, gli = _cmm_big(xr, None, wl[0], wl[1])
        grr, gri = _cmm_big(xr, None, wr[0], wr[1])
        resr, resi = _cmm_big(xr, None, wres[0], wres[1])
    else:
        glr, gli = _cmm_big(xr, xi, wl[0], wl[1])
        grr, gri = _cmm_big(xr, xi, wr[0], wr[1])
        resr, resi = _cmm_big(xr, xi, wres[0], wres[1])
    slr, sli, srr, sri = _sc_gather4(glr, gli, grr, gri, src, dst)
    eye = jnp.eye(HEADS, dtype=_F32)
    a0bd = (eye[:, None, :] * att[0][:, :, None]).reshape(HEADS * ch, HEADS)
    a1bdn = (eye[:, None, :] * (-att[1])[:, :, None]).reshape(HEADS * ch, HEADS)
    bh = jnp.repeat(eye, ch, axis=1)
    er = _er(slr, sli, srr, sri, a0bd, a1bdn)
    ex = _softnum(er)
    den = _den(ex, dst2d)
    aggr, aggi = _agg(ex, slr, sli, dst2d, bh)
    return _post(aggr, aggi, den, bh, resr, resi, bias.reshape(2, HEADS * ch))


def kernel(x, edge_index, Wl1, Wr1, a1, b1, Wres1, Wl2, Wr2, a2, b2, Wres2,
           l1W, l1b, g1, be1, l2W, l2b, g2, be2, l3W, l3b):
    src = edge_index[0]
    dst = edge_index[1]
    dst2d = dst.reshape(1, N_EDGES)
    h1r, h1i = _gat_layer(x, None, src, dst, dst2d, Wl1, Wr1, a1, b1, Wres1, 64)
    h2r, h2i = _gat_layer(h1r, h1i, src, dst, dst2d, Wl2, Wr2, a2, b2, Wres2, 512)
    m1r, m1i = _cmm_mlp(h2r, h2i, l1W[0], l1W[1], l1b, "bn_relu", g1, be1)
    m2r, m2i = _cmm_mlp(m1r, m1i, l2W[0], l2W[1], l2b, "bn_relu", g2, be2)
    outr, outi = _cmm_mlp(m2r, m2i, l3W[0], l3W[1], l3b, "norm")
    return jnp.stack([outr, outi])


# speedup vs baseline: 3.2099x; 1.0399x over previous
import functools

import jax
import jax.numpy as jnp
from jax import lax
from jax.experimental import pallas as pl
from jax.experimental.pallas import tpu as pltpu
from jax.experimental.pallas import tpu_sc as plsc

N_NODES = 1024
N_EDGES = 8192
HEADS = 20

_F32 = jnp.float32
_DN_T = (((1,), (1,)), ((), ()))
_DN_N = (((1,), (0,)), ((), ()))


def _blk(dim, target):
    if dim <= target:
        return dim
    b = (target // 128) * 128
    while b > 0 and dim % b:
        b -= 128
    return b if b > 0 else dim


def _sc_gather_rows(table, idx):
    n, d = table.shape
    e = idx.shape[0]
    nw = 32
    bpw = e // nw
    ch = max(8, ((96 * 1024) // (d * 4)) // 8 * 8)
    while ch > 8 and (bpw % ch or (bpw // ch) % 2):
        ch -= 8
    nbuf = 2 if 2 * ch * d + bpw <= 120000 else 1
    n_chunks = bpw // ch
    mesh = plsc.VectorSubcoreMesh(core_axis_name="c", subcore_axis_name="s")
    scratch = (
        [pltpu.VMEM((bpw,), jnp.int32)]
        + [pltpu.VMEM((ch, d), _F32) for _ in range(nbuf)]
        + [pltpu.SemaphoreType.DMA for _ in range(nbuf)]
    )

    @functools.partial(pl.kernel,
                       out_type=jax.ShapeDtypeStruct((e, d), _F32),
                       mesh=mesh, scratch_types=scratch)
    def gk(tab_h, idx_h, out_h, idx_v, *bufsem):
        bufs = bufsem[:nbuf]
        sems = bufsem[nbuf:]
        wid = lax.axis_index("s") * 2 + lax.axis_index("c")
        base = wid * bpw
        pltpu.sync_copy(idx_h.at[pl.ds(base, bpw)], idx_v)

        def gather(g, slot):
            pltpu.make_async_copy(
                tab_h.at[idx_v.at[pl.ds(g * ch, ch)]], bufs[slot], sems[slot]
            ).start()

        def drain(g, slot):
            pltpu.make_async_copy(
                tab_h.at[idx_v.at[pl.ds(g * ch, ch)]], bufs[slot], sems[slot]
            ).wait()
            pltpu.sync_copy(bufs[slot], out_h.at[pl.ds(base + g * ch, ch)])

        if nbuf == 1:
            def body1(g, carry):
                gather(g, 0)
                drain(g, 0)
                return carry

            lax.fori_loop(0, n_chunks, body1, 0)
        else:
            n_pairs = n_chunks // 2
            gather(0, 0)

            def body2(h, carry):
                g0 = 2 * h
                gather(g0 + 1, 1)
                drain(g0, 0)

                @pl.when(h + 1 < n_pairs)
                def _():
                    gather(g0 + 2, 0)

                drain(g0 + 1, 1)
                return carry

            lax.fori_loop(0, n_pairs, body2, 0)

    return gk(table, idx)


def _sc_gather4(glr, gli, grr, gri, src, dst):
    return (_sc_gather_rows(glr, src), _sc_gather_rows(gli, src),
            _sc_gather_rows(grr, dst), _sc_gather_rows(gri, dst))


def _sc_build_adj(ext, src, dst, zrow):
    e = src.shape[0]
    rows = N_NODES // 16
    hpc = HEADS // 2
    mesh = plsc.VectorSubcoreMesh(core_axis_name="c", subcore_axis_name="s")
    scratch = [
        pltpu.VMEM((e,), jnp.int32),
        pltpu.VMEM((e,), jnp.int32),
        pltpu.VMEM((e,), _F32),
        pltpu.VMEM((rows * N_NODES,), _F32),
    ]

    @functools.partial(pl.kernel,
                       out_type=jax.ShapeDtypeStruct(
                           (HEADS, N_NODES * N_NODES), _F32),
                       mesh=mesh, scratch_types=scratch,
                       compiler_params=pltpu.CompilerParams(
                           needs_layout_passes=False))
    def bk(ext_h, src_h, dst_h, z_h, a_h, src_v, dst_v, ex_v, abuf):
        cid = lax.axis_index("c")
        sid = lax.axis_index("s")
        base = sid * rows
        pltpu.sync_copy(src_h, src_v)
        pltpu.sync_copy(dst_h, dst_v)

        def per_head(j, carry):
            h = j * 2 + cid
            pltpu.sync_copy(ext_h.at[h], ex_v)
            pltpu.sync_copy(z_h, abuf)

            def per_vec(v, carry2):
                sl = pl.ds(v * 16, 16)
                d16 = dst_v[sl]
                s16 = src_v[sl]
                x16 = ex_v[sl]
                rel = d16 - base
                msk = (rel >= 0) & (rel < rows)
                flat = jnp.where(msk, rel * N_NODES + s16, 0)
                plsc.addupdate_scatter(abuf, [flat], x16, mask=msk)
                return carry2

            lax.fori_loop(0, e // 16, per_vec, 0)
            pltpu.sync_copy(abuf, a_h.at[h, pl.ds(base * N_NODES,
                                                  rows * N_NODES)])
            return carry

        lax.fori_loop(0, hpc, per_head, 0)

    return bk(ext, src, dst, zrow).reshape(HEADS, N_NODES, N_NODES)


def _agg_adj(a, glr, gli):
    n = N_NODES
    hc = glr.shape[1]
    c = hc // HEADS

    def body(a_ref, glr_ref, gli_ref, or_ref, oi_ref):
        am = a_ref[0]
        or_ref[...] = lax.dot_general(am, glr_ref[...], _DN_N,
                                      preferred_element_type=_F32)
        oi_ref[...] = lax.dot_general(am, gli_ref[...], _DN_N,
                                      preferred_element_type=_F32)

    return pl.pallas_call(
        body,
        grid=(HEADS,),
        in_specs=[
            pl.BlockSpec((1, n, n), lambda h: (h, 0, 0)),
            pl.BlockSpec((n, c), lambda h: (0, h)),
            pl.BlockSpec((n, c), lambda h: (0, h)),
        ],
        out_specs=[pl.BlockSpec((n, c), lambda h: (0, h))] * 2,
        out_shape=[jax.ShapeDtypeStruct((n, hc), _F32)] * 2,
        compiler_params=pltpu.CompilerParams(
            dimension_semantics=("arbitrary",)),
    )(a, glr, gli)


def _cmm_big(xr, xi, w0, w1, bias=None, relu=False):
    m, k_dim = xr.shape
    nout = w0.shape[0]
    bm = _blk(m, 256)
    bn = _blk(nout, 512)
    bk = _blk(k_dim, 2048)
    gm, gn, gk = m // bm, nout // bn, k_dim // bk
    has_xi = xi is not None
    has_b = bias is not None

    def body(*refs):
        i = 0
        xr_ref = refs[i]; i += 1
        xi_ref = None
        if has_xi:
            xi_ref = refs[i]; i += 1
        w0_ref = refs[i]; w1_ref = refs[i + 1]; i += 2
        b_ref = None
        if has_b:
            b_ref = refs[i]; i += 1
        or_ref, oi_ref, accr, acci = refs[i:i + 4]
        kk = pl.program_id(2)

        @pl.when(kk == 0)
        def _():
            accr[...] = jnp.zeros(accr.shape, _F32)
            acci[...] = jnp.zeros(acci.shape, _F32)

        xrb = xr_ref[...]
        w0b = w0_ref[...]
        w1b = w1_ref[...]
        pr = lax.dot_general(xrb, w0b, _DN_T, preferred_element_type=_F32)
        pi = lax.dot_general(xrb, w1b, _DN_T, preferred_element_type=_F32)
        if has_xi:
            xib = xi_ref[...]
            pr = pr - lax.dot_general(xib, w1b, _DN_T, preferred_element_type=_F32)
            pi = pi + lax.dot_general(xib, w0b, _DN_T, preferred_element_type=_F32)
        accr[...] += pr
        acci[...] += pi

        @pl.when(kk == gk - 1)
        def _():
            yr = accr[...]
            yi = acci[...]
            if has_b:
                yr = yr + b_ref[0]
                yi = yi + b_ref[1]
            if relu:
                yr = jnp.maximum(yr, 0.0)
                yi = jnp.maximum(yi, 0.0)
            or_ref[...] = yr
            oi_ref[...] = yi

    in_specs = [pl.BlockSpec((bm, bk), lambda m_, n_, k_: (m_, k_))]
    ops = [xr]
    if has_xi:
        in_specs.append(pl.BlockSpec((bm, bk), lambda m_, n_, k_: (m_, k_)))
        ops.append(xi)
    in_specs += [pl.BlockSpec((bn, bk), lambda m_, n_, k_: (n_, k_))] * 2
    ops += [w0, w1]
    if has_b:
        in_specs.append(pl.BlockSpec((2, bn), lambda m_, n_, k_: (0, n_)))
        ops.append(bias)
    yr, yi = pl.pallas_call(
        body,
        grid=(gm, gn, gk),
        in_specs=in_specs,
        out_specs=[pl.BlockSpec((bm, bn), lambda m_, n_, k_: (m_, n_))] * 2,
        out_shape=[jax.ShapeDtypeStruct((m, nout), _F32)] * 2,
        scratch_shapes=[pltpu.VMEM((bm, bn), _F32)] * 2,
        compiler_params=pltpu.CompilerParams(
            dimension_semantics=("parallel", "parallel", "arbitrary")),
    )(*ops)
    return yr, yi


def _cmm_mlp(xr, xi, w0, w1, bias, mode, g=None, be=None):
    m, k_dim = xr.shape
    nout = w0.shape[0]
    bk = _blk(k_dim, 1024)
    gk = k_dim // bk

    def body(*refs):
        i = 0
        xr_ref, xi_ref, w0_ref, w1_ref, b_ref = refs[:5]
        i = 5
        g_ref = be_ref = None
        if mode == "bn_relu":
            g_ref, be_ref = refs[i:i + 2]
            i += 2
        or_ref, oi_ref, accr, acci = refs[i:i + 4]
        kk = pl.program_id(0)

        @pl.when(kk == 0)
        def _():
            accr[...] = jnp.zeros(accr.shape, _F32)
            acci[...] = jnp.zeros(acci.shape, _F32)

        xrb = xr_ref[...]
        xib = xi_ref[...]
        w0b = w0_ref[...]
        w1b = w1_ref[...]
        accr[...] += (lax.dot_general(xrb, w0b, _DN_T, preferred_element_type=_F32)
                      - lax.dot_general(xib, w1b, _DN_T, preferred_element_type=_F32))
        acci[...] += (lax.dot_general(xrb, w1b, _DN_T, preferred_element_type=_F32)
                      + lax.dot_general(xib, w0b, _DN_T, preferred_element_type=_F32))

        @pl.when(kk == gk - 1)
        def _():
            yr = accr[...] + b_ref[0]
            yi = acci[...] + b_ref[1]
            if mode == "bn_relu":
                mr = jnp.mean(yr, axis=0, keepdims=True)
                vr = jnp.mean((yr - mr) ** 2, axis=0, keepdims=True)
                yr = (yr - mr) * lax.rsqrt(vr + 1e-5) * g_ref[0] + be_ref[0]
                mi = jnp.mean(yi, axis=0, keepdims=True)
                vi = jnp.mean((yi - mi) ** 2, axis=0, keepdims=True)
                yi = (yi - mi) * lax.rsqrt(vi + 1e-5) * g_ref[1] + be_ref[1]
                yr = jnp.maximum(yr, 0.0)
                yi = jnp.maximum(yi, 0.0)
            elif mode == "norm":
                nrm = jnp.sqrt(jnp.sum(yr * yr + yi * yi, axis=1, keepdims=True))
                dnm = jnp.maximum(nrm, 1.0)
                yr = yr / dnm
                yi = yi / dnm
            or_ref[...] = yr
            oi_ref[...] = yi

    in_specs = [
        pl.BlockSpec((m, bk), lambda k_: (0, k_)),
        pl.BlockSpec((m, bk), lambda k_: (0, k_)),
        pl.BlockSpec((nout, bk), lambda k_: (0, k_)),
        pl.BlockSpec((nout, bk), lambda k_: (0, k_)),
        pl.BlockSpec((2, nout), lambda k_: (0, 0)),
    ]
    ops = [xr, xi, w0, w1, bias]
    if mode == "bn_relu":
        in_specs += [pl.BlockSpec((2, nout), lambda k_: (0, 0))] * 2
        ops += [g, be]
    yr, yi = pl.pallas_call(
        body,
        grid=(gk,),
        in_specs=in_specs,
        out_specs=[pl.BlockSpec((m, nout), lambda k_: (0, 0))] * 2,
        out_shape=[jax.ShapeDtypeStruct((m, nout), _F32)] * 2,
        scratch_shapes=[pltpu.VMEM((m, nout), _F32)] * 2,
        compiler_params=pltpu.CompilerParams(
            dimension_semantics=("arbitrary",)),
    )(*ops)
    return yr, yi


def _er(slr, sli, srr, sri, a0bd, a1bdn):
    e, hc = slr.shape
    be = 512
    bk = _blk(hc, 2560)
    ge, gk = e // be, hc // bk

    def body(slr_ref, sli_ref, srr_ref, sri_ref, a0_ref, a1_ref, out_ref, acc):
        kk = pl.program_id(1)

        @pl.when(kk == 0)
        def _():
            acc[...] = jnp.zeros(acc.shape, _F32)

        hr = jnp.maximum(slr_ref[...] + srr_ref[...], 0.0)
        hi = jnp.maximum(sli_ref[...] + sri_ref[...], 0.0)
        acc[...] += (lax.dot_general(hr, a0_ref[...], _DN_N, preferred_element_type=_F32)
                     + lax.dot_general(hi, a1_ref[...], _DN_N, preferred_element_type=_F32))

        @pl.when(kk == gk - 1)
        def _():
            out_ref[...] = acc[...]

    return pl.pallas_call(
        body,
        grid=(ge, gk),
        in_specs=[pl.BlockSpec((be, bk), lambda e_, k_: (e_, k_))] * 4
        + [pl.BlockSpec((bk, HEADS), lambda e_, k_: (k_, 0))] * 2,
        out_specs=pl.BlockSpec((be, HEADS), lambda e_, k_: (e_, 0)),
        out_shape=jax.ShapeDtypeStruct((e, HEADS), _F32),
        scratch_shapes=[pltpu.VMEM((be, HEADS), _F32)],
        compiler_params=pltpu.CompilerParams(
            dimension_semantics=("parallel", "arbitrary")),
    )(slr, sli, srr, sri, a0bd, a1bdn)


def _softnum(er):
    def body(er_ref, ex_ref, ext_ref):
        erb = er_ref[...]
        mg = jnp.max(erb, axis=0, keepdims=True)
        exb = jnp.exp(erb - mg)
        ex_ref[...] = exb
        ext_ref[...] = exb.T

    return pl.pallas_call(
        body,
        out_shape=[jax.ShapeDtypeStruct(er.shape, _F32),
                   jax.ShapeDtypeStruct(er.shape[::-1], _F32)],
    )(er)


def _den(ex, dst2d):
    e, h = ex.shape
    be = 512
    ge = e // be

    def body(dst_ref, ex_ref, out_ref, acc):
        ee = pl.program_id(0)

        @pl.when(ee == 0)
        def _():
            acc[...] = jnp.zeros(acc.shape, _F32)

        iota = lax.broadcasted_iota(jnp.int32, (N_NODES, be), 0)
        oh = (iota == dst_ref[...]).astype(_F32)
        acc[...] += lax.dot_general(oh, ex_ref[...], _DN_N,
                                    preferred_element_type=_F32)

        @pl.when(ee == ge - 1)
        def _():
            out_ref[...] = acc[...]

    return pl.pallas_call(
        body,
        grid=(ge,),
        in_specs=[
            pl.BlockSpec((1, be), lambda e_: (0, e_)),
            pl.BlockSpec((be, h), lambda e_: (e_, 0)),
        ],
        out_specs=pl.BlockSpec((N_NODES, h), lambda e_: (0, 0)),
        out_shape=jax.ShapeDtypeStruct((N_NODES, h), _F32),
        scratch_shapes=[pltpu.VMEM((N_NODES, h), _F32)],
        compiler_params=pltpu.CompilerParams(
            dimension_semantics=("arbitrary",)),
    )(dst2d, ex)


def _agg(ex, slr, sli, dst2d, bh):
    e, hc = slr.shape
    bc = _blk(hc, 512)
    be = 512
    gc, ge = hc // bc, e // be

    def body(dst_ref, ex_ref, bh_ref, slr_ref, sli_ref, or_ref, oi_ref,
             accr, acci):
        ee = pl.program_id(1)

        @pl.when(ee == 0)
        def _():
            accr[...] = jnp.zeros(accr.shape, _F32)
            acci[...] = jnp.zeros(acci.shape, _F32)

        iota = lax.broadcasted_iota(jnp.int32, (N_NODES, be), 0)
        oh = (iota == dst_ref[...]).astype(_F32)
        exc = lax.dot_general(ex_ref[...], bh_ref[...], _DN_N,
                              preferred_element_type=_F32)
        accr[...] += lax.dot_general(oh, exc * slr_ref[...], _DN_N,
                                     preferred_element_type=_F32)
        acci[...] += lax.dot_general(oh, exc * sli_ref[...], _DN_N,
                                     preferred_element_type=_F32)

        @pl.when(ee == ge - 1)
        def _():
            or_ref[...] = accr[...]
            oi_ref[...] = acci[...]

    return pl.pallas_call(
        body,
        grid=(gc, ge),
        in_specs=[
            pl.BlockSpec((1, be), lambda c_, e_: (0, e_)),
            pl.BlockSpec((be, HEADS), lambda c_, e_: (e_, 0)),
            pl.BlockSpec((HEADS, bc), lambda c_, e_: (0, c_)),
            pl.BlockSpec((be, bc), lambda c_, e_: (e_, c_)),
            pl.BlockSpec((be, bc), lambda c_, e_: (e_, c_)),
        ],
        out_specs=[pl.BlockSpec((N_NODES, bc), lambda c_, e_: (0, c_))] * 2,
        out_shape=[jax.ShapeDtypeStruct((N_NODES, hc), _F32)] * 2,
        scratch_shapes=[pltpu.VMEM((N_NODES, bc), _F32)] * 2,
        compiler_params=pltpu.CompilerParams(
            dimension_semantics=("parallel", "arbitrary")),
    )(dst2d, ex, bh, slr, sli)


def _post(aggr, aggi, den, bh, resr, resi, bias):
    n, hc = aggr.shape
    bc = _blk(hc, 512)
    gc = hc // bc

    def body(aggr_ref, aggi_ref, den_ref, bh_ref, resr_ref, resi_ref, b_ref,
             or_ref, oi_ref):
        dexp = lax.dot_general(den_ref[...], bh_ref[...], _DN_N,
                               preferred_element_type=_F32)
        r = 1.0 / (dexp + 1e-30)
        or_ref[...] = jnp.maximum(aggr_ref[...] * r + resr_ref[...] + b_ref[0], 0.0)
        oi_ref[...] = jnp.maximum(aggi_ref[...] * r + resi_ref[...] + b_ref[1], 0.0)

    return pl.pallas_call(
        body,
        grid=(gc,),
        in_specs=[
            pl.BlockSpec((n, bc), lambda c_: (0, c_)),
            pl.BlockSpec((n, bc), lambda c_: (0, c_)),
            pl.BlockSpec((n, HEADS), lambda c_: (0, 0)),
            pl.BlockSpec((HEADS, bc), lambda c_: (0, c_)),
            pl.BlockSpec((n, bc), lambda c_: (0, c_)),
            pl.BlockSpec((n, bc), lambda c_: (0, c_)),
            pl.BlockSpec((2, bc), lambda c_: (0, c_)),
        ],
        out_specs=[pl.BlockSpec((n, bc), lambda c_: (0, c_))] * 2,
        out_shape=[jax.ShapeDtypeStruct((n, hc), _F32)] * 2,
        compiler_params=pltpu.CompilerParams(
            dimension_semantics=("parallel",)),
    )(aggr, aggi, den, bh, resr, resi, bias)


def _gat_layer(xr, xi, src, dst, dst2d, wl, wr, att, bias, wres, ch):
    if xi is None:
        glr, gli = _cmm_big(xr, None, wl[0], wl[1])
        grr, gri = _cmm_big(xr, None, wr[0], wr[1])
        resr, resi = _cmm_big(xr, None, wres[0], wres[1])
    else:
        glr, gli = _cmm_big(xr, xi, wl[0], wl[1])
        grr, gri = _cmm_big(xr, xi, wr[0], wr[1])
        resr, resi = _cmm_big(xr, xi, wres[0], wres[1])
    slr, sli, srr, sri = _sc_gather4(glr, gli, grr, gri, src, dst)
    eye = jnp.eye(HEADS, dtype=_F32)
    a0bd = (eye[:, None, :] * att[0][:, :, None]).reshape(HEADS * ch, HEADS)
    a1bdn = (eye[:, None, :] * (-att[1])[:, :, None]).reshape(HEADS * ch, HEADS)
    bh = jnp.repeat(eye, ch, axis=1)
    er = _er(slr, sli, srr, sri, a0bd, a1bdn)
    ex, ext = _softnum(er)
    den = _den(ex, dst2d)
    if ch % 128 == 0:
        zrow = jnp.zeros(((N_NODES // 16) * N_NODES,), _F32)
        adj = _sc_build_adj(ext, src, dst, zrow)
        aggr, aggi = _agg_adj(adj, glr, gli)
    else:
        aggr, aggi = _agg(ex, slr, sli, dst2d, bh)
    return _post(aggr, aggi, den, bh, resr, resi, bias.reshape(2, HEADS * ch))


def kernel(x, edge_index, Wl1, Wr1, a1, b1, Wres1, Wl2, Wr2, a2, b2, Wres2,
           l1W, l1b, g1, be1, l2W, l2b, g2, be2, l3W, l3b):
    src = edge_index[0]
    dst = edge_index[1]
    dst2d = dst.reshape(1, N_EDGES)
    h1r, h1i = _gat_layer(x, None, src, dst, dst2d, Wl1, Wr1, a1, b1, Wres1, 64)
    h2r, h2i = _gat_layer(h1r, h1i, src, dst, dst2d, Wl2, Wr2, a2, b2, Wres2, 512)
    m1r, m1i = _cmm_mlp(h2r, h2i, l1W[0], l1W[1], l1b, "bn_relu", g1, be1)
    m2r, m2i = _cmm_mlp(m1r, m1i, l2W[0], l2W[1], l2b, "bn_relu", g2, be2)
    outr, outi = _cmm_mlp(m2r, m2i, l3W[0], l3W[1], l3b, "norm")
    return jnp.stack([outr, outi])
